# baseline (speedup 1.0000x reference)
import functools

import jax
import jax.numpy as jnp
from jax import lax
from jax.experimental import pallas as pl
from jax.experimental.pallas import tpu as pltpu
from jax.experimental.pallas import tpu_sc as plsc

N = 100000
D = 128
S = 1024
NC = 2
NS = 16
NW = NC * NS
L = 96
J = 2
P = L * J
NP = N // P
K = -(-NP // NW)
NITER = 3 * (-(-(K + 2) // 3))
TAIL = N - NP * P
TAIL_C = 80
TBASE = NP * P
SEG_W = S // NS
SEG_H = SEG_W // 2

_mesh = plsc.VectorSubcoreMesh(core_axis_name="c", subcore_axis_name="s")


def _worker_ids():
    cid = lax.axis_index("c")
    sid = lax.axis_index("s")
    return cid, sid, sid * NC + cid


@functools.partial(
    pl.kernel,
    out_type=[
        jax.ShapeDtypeStruct((NC * S, D), jnp.float32),
        jax.ShapeDtypeStruct((NC * S, D), jnp.float32),
    ],
    mesh=_mesh,
    scratch_types=[
        pltpu.VMEM_SHARED((S, D), jnp.float32),
        pltpu.VMEM_SHARED((S, D), jnp.float32),
        pltpu.VMEM((SEG_H, D), jnp.float32),
        pltpu.VMEM((SEG_H, D), jnp.float32),
        pltpu.VMEM((L, D), jnp.float32),
        [pltpu.VMEM((P, D), jnp.float32)] * 3,
        [pltpu.VMEM((J, L), jnp.int32)] * 3,
        pltpu.VMEM((TAIL_C,), jnp.int32),
        [pltpu.SemaphoreType.DMA] * 3,
        [pltpu.SemaphoreType.DMA] * 3,
    ],
)
def _seg_sums(x_hbm, b_hbm, sums_hbm, cnts_hbm,
              sh_sum, sh_cnt, zbuf, zcbuf, ones, xb, ib, ibt, ins, scs):
    cid, sid, wid = _worker_ids()

    zero = jnp.zeros((16,), jnp.float32)
    one = jnp.ones((16,), jnp.float32)

    def _zrow(r, _):
        for j in range(D // 16):
            ds = pl.ds(j * 16, 16)
            zbuf[r, ds] = zero
            zcbuf[r, ds] = zero
        return 0

    lax.fori_loop(0, SEG_H, _zrow, 0)

    def _orow(r, _):
        for j in range(D // 16):
            ones[r, pl.ds(j * 16, 16)] = one
        return 0

    lax.fori_loop(0, L, _orow, 0)

    for half in range(2):
        hb = sid * SEG_W + half * SEG_H
        pltpu.sync_copy(zbuf, sh_sum.at[pl.ds(hb, SEG_H)])
        pltpu.sync_copy(zcbuf, sh_cnt.at[pl.ds(hb, SEG_H)])
    plsc.subcore_barrier()

    def _issue_in(p, h):
        base = p * P
        pltpu.async_copy(x_hbm.at[pl.ds(base, P)], xb[h], ins[h])
        for j in range(J):
            pltpu.async_copy(b_hbm.at[pl.ds(base + j * L, L)], ib[h].at[j], ins[h])

    def _wait_in(p, h):
        base = p * P
        pltpu.make_async_copy(x_hbm.at[pl.ds(base, P)], xb[h], ins[h]).wait()
        for j in range(J):
            pltpu.make_async_copy(b_hbm.at[pl.ds(base + j * L, L)], ib[h].at[j], ins[h]).wait()

    def _issue_scatter(h):
        for j in range(J):
            pltpu.async_copy(xb[h].at[pl.ds(j * L, L)], sh_sum.at[ib[h].at[j]], scs[h], add=True)
            pltpu.async_copy(ones, sh_cnt.at[ib[h].at[j]], scs[h], add=True)

    def _wait_scatter(h):
        for j in range(J):
            pltpu.make_async_copy(xb[h].at[pl.ds(j * L, L)], sh_sum.at[ib[h].at[j]], scs[h]).wait()
            pltpu.make_async_copy(ones, sh_cnt.at[ib[h].at[j]], scs[h]).wait()

    _issue_in(wid, 0)

    def _trip(kk, _):
        for h in range(3):
            k = kk * 3 + h
            p = k * NW + wid

            @pl.when(p < NP)
            def _():
                _wait_in(p, h)
                _issue_scatter(h)

            h2 = (h + 1) % 3
            pp2 = p - 2 * NW

            @pl.when(jnp.logical_and(pp2 >= 0, pp2 < NP))
            def _():
                _wait_scatter(h2)

            pn = p + NW

            @pl.when(pn < NP)
            def _():
                _issue_in(pn, h2)

        return 0

    lax.fori_loop(0, NITER // 3, _trip, 0)

    @pl.when(wid == NW - 1)
    def _():
        for t in range(TAIL // TAIL_C):
            off = TBASE + t * TAIL_C
            pltpu.sync_copy(x_hbm.at[pl.ds(off, TAIL_C)], xb[0].at[pl.ds(0, TAIL_C)])
            pltpu.sync_copy(b_hbm.at[pl.ds(off, TAIL_C)], ibt)
            pltpu.sync_copy(xb[0].at[pl.ds(0, TAIL_C)], sh_sum.at[ibt], add=True)
            pltpu.sync_copy(ones.at[pl.ds(0, TAIL_C)], sh_cnt.at[ibt], add=True)

    plsc.subcore_barrier()

    for half in range(2):
        hb = sid * SEG_W + half * SEG_H
        pltpu.sync_copy(sh_sum.at[pl.ds(hb, SEG_H)], zbuf)
        pltpu.sync_copy(zbuf, sums_hbm.at[pl.ds(cid * S + hb, SEG_H)])
        pltpu.sync_copy(sh_cnt.at[pl.ds(hb, SEG_H)], zcbuf)
        pltpu.sync_copy(zcbuf, cnts_hbm.at[pl.ds(cid * S + hb, SEG_H)])


@functools.partial(
    pl.kernel,
    out_type=jax.ShapeDtypeStruct((N, D), jnp.float32),
    mesh=_mesh,
    scratch_types=[
        pltpu.VMEM_SHARED((S, D), jnp.float32),
        pltpu.VMEM((SEG_H, D), jnp.float32),
        pltpu.VMEM((SEG_H, D), jnp.float32),
        pltpu.VMEM((SEG_H, D), jnp.float32),
        pltpu.VMEM((SEG_H, D), jnp.float32),
        [pltpu.VMEM((P, D), jnp.float32)] * 3,
        [pltpu.VMEM((J, L), jnp.int32)] * 3,
        pltpu.VMEM((TAIL_C,), jnp.int32),
        [pltpu.SemaphoreType.DMA] * 3,
        [pltpu.SemaphoreType.DMA] * 3,
        [pltpu.SemaphoreType.DMA] * 3,
    ],
)
def _subtract(x_hbm, b_hbm, sums_hbm, cnts_hbm, y_hbm,
              sh_nmean, s0, s1, c0, c1, xb, ib, ibt, ins, gs, os):
    cid, sid, wid = _worker_ids()

    for half in range(2):
        base = sid * SEG_W + half * SEG_H
        pltpu.sync_copy(sums_hbm.at[pl.ds(base, SEG_H)], s0)
        pltpu.sync_copy(sums_hbm.at[pl.ds(S + base, SEG_H)], s1)
        pltpu.sync_copy(cnts_hbm.at[pl.ds(base, SEG_H)], c0)
        pltpu.sync_copy(cnts_hbm.at[pl.ds(S + base, SEG_H)], c1)

        def _mrow(r, _):
            ds0 = pl.ds(0, 16)
            cnt = c0[r, ds0] + c1[r, ds0]
            ninv = jnp.float32(-1.0) / jnp.maximum(cnt, jnp.float32(1.0))
            for j in range(D // 16):
                ds = pl.ds(j * 16, 16)
                s0[r, ds] = (s0[r, ds] + s1[r, ds]) * ninv
            return 0

        lax.fori_loop(0, SEG_H, _mrow, 0)
        pltpu.sync_copy(s0, sh_nmean.at[pl.ds(base, SEG_H)])

    plsc.subcore_barrier()

    def _issue_in(p, h):
        base = p * P
        pltpu.async_copy(x_hbm.at[pl.ds(base, P)], xb[h], ins[h])
        for j in range(J):
            pltpu.async_copy(b_hbm.at[pl.ds(base + j * L, L)], ib[h].at[j], ins[h])

    def _wait_in(p, h):
        base = p * P
        pltpu.make_async_copy(x_hbm.at[pl.ds(base, P)], xb[h], ins[h]).wait()
        for j in range(J):
            pltpu.make_async_copy(b_hbm.at[pl.ds(base + j * L, L)], ib[h].at[j], ins[h]).wait()

    def _issue_gadd(h):
        for j in range(J):
            pltpu.async_copy(sh_nmean.at[ib[h].at[j]], xb[h].at[pl.ds(j * L, L)], gs[h], add=True)

    def _wait_gadd(h):
        for j in range(J):
            pltpu.make_async_copy(sh_nmean.at[ib[h].at[j]], xb[h].at[pl.ds(j * L, L)], gs[h]).wait()

    _issue_in(wid, 0)

    def _trip(kk, _):
        for h in range(3):
            k = kk * 3 + h
            p = k * NW + wid

            @pl.when(p < NP)
            def _():
                _wait_in(p, h)
                _issue_gadd(h)

            h1 = (h + 2) % 3
            pp1 = p - NW

            @pl.when(jnp.logical_and(pp1 >= 0, pp1 < NP))
            def _():
                _wait_gadd(h1)
                pltpu.async_copy(xb[h1], y_hbm.at[pl.ds(pp1 * P, P)], os[h1])

            h2 = (h + 1) % 3
            pp2 = p - 2 * NW

            @pl.when(jnp.logical_and(pp2 >= 0, pp2 < NP))
            def _():
                pltpu.make_async_copy(xb[h2], y_hbm.at[pl.ds(pp2 * P, P)], os[h2]).wait()

            pn = p + NW

            @pl.when(pn < NP)
            def _():
                _issue_in(pn, h2)

        return 0

    lax.fori_loop(0, NITER // 3, _trip, 0)

    @pl.when(wid == NW - 1)
    def _():
        for t in range(TAIL // TAIL_C):
            off = TBASE + t * TAIL_C
            pltpu.sync_copy(x_hbm.at[pl.ds(off, TAIL_C)], xb[0].at[pl.ds(0, TAIL_C)])
            pltpu.sync_copy(b_hbm.at[pl.ds(off, TAIL_C)], ibt)
            pltpu.sync_copy(sh_nmean.at[ibt], xb[0].at[pl.ds(0, TAIL_C)], add=True)
            pltpu.sync_copy(xb[0].at[pl.ds(0, TAIL_C)], y_hbm.at[pl.ds(off, TAIL_C)])


def kernel(x, batch, dim_size):
    del dim_size
    b32 = batch.astype(jnp.int32)
    sums, cnts = _seg_sums(x, b32)
    return _subtract(x, b32, sums, cnts)


# speedup vs baseline: 1.1068x; 1.1068x over previous
import functools

import jax
import jax.numpy as jnp
from jax import lax
from jax.experimental import pallas as pl
from jax.experimental.pallas import tpu as pltpu
from jax.experimental.pallas import tpu_sc as plsc

N = 100000
D = 128
S = 1024
NC = 2
NS = 16
NW = NC * NS
P = 128
NP = N // P
K = -(-NP // NW)
NITER = K + 2
TAIL = N - NP * P
TBASE = NP * P
SEG_W = S // NS

_mesh = plsc.VectorSubcoreMesh(core_axis_name="c", subcore_axis_name="s")


def _worker_ids():
    cid = lax.axis_index("c")
    sid = lax.axis_index("s")
    return cid, sid, sid * NC + cid


@functools.partial(
    pl.kernel,
    out_type=[
        jax.ShapeDtypeStruct((NC * S, D), jnp.float32),
        jax.ShapeDtypeStruct((NC * S, D), jnp.float32),
    ],
    mesh=_mesh,
    scratch_types=[
        pltpu.VMEM_SHARED((S, D), jnp.float32),
        pltpu.VMEM_SHARED((S, D), jnp.float32),
        pltpu.VMEM((SEG_W, D), jnp.float32),
        pltpu.VMEM((SEG_W, D), jnp.float32),
        pltpu.VMEM((P, D), jnp.float32),
        [pltpu.VMEM((P, D), jnp.float32)] * 3,
        [pltpu.VMEM((P,), jnp.int32)] * 3,
        pltpu.VMEM((TAIL, D), jnp.float32),
        pltpu.VMEM((TAIL,), jnp.int32),
        [pltpu.SemaphoreType.DMA] * 3,
        [pltpu.SemaphoreType.DMA] * 3,
    ],
)
def _seg_sums(x_hbm, b_hbm, sums_hbm, cnts_hbm,
              sh_sum, sh_cnt, zbuf, zcbuf, ones, xb, ib, xbt, ibt, ins, scs):
    cid, sid, wid = _worker_ids()

    zero = jnp.zeros((16,), jnp.float32)
    one = jnp.ones((16,), jnp.float32)

    def _zrow(r, _):
        for j in range(D // 16):
            ds = pl.ds(j * 16, 16)
            zbuf[r, ds] = zero
            zcbuf[r, ds] = zero
        return 0

    lax.fori_loop(0, SEG_W, _zrow, 0)

    def _orow(r, _):
        for j in range(D // 16):
            ones[r, pl.ds(j * 16, 16)] = one
        return 0

    lax.fori_loop(0, P, _orow, 0)

    pltpu.sync_copy(zbuf, sh_sum.at[pl.ds(sid * SEG_W, SEG_W)])
    pltpu.sync_copy(zcbuf, sh_cnt.at[pl.ds(sid * SEG_W, SEG_W)])
    plsc.subcore_barrier()

    pltpu.async_copy(x_hbm.at[pl.ds(wid * P, P)], xb[0], ins[0])
    pltpu.async_copy(b_hbm.at[pl.ds(wid * P, P)], ib[0], ins[0])

    def _trip(kk, _):
        for h in range(3):
            k = kk * 3 + h
            p = k * NW + wid

            @pl.when(p < NP)
            def _():
                base = p * P
                pltpu.make_async_copy(x_hbm.at[pl.ds(base, P)], xb[h], ins[h]).wait()
                pltpu.make_async_copy(b_hbm.at[pl.ds(base, P)], ib[h], ins[h]).wait()
                pltpu.async_copy(xb[h], sh_sum.at[ib[h]], scs[h], add=True)
                pltpu.async_copy(ones, sh_cnt.at[ib[h]], scs[h], add=True)

            h2 = (h + 1) % 3
            pp2 = p - 2 * NW

            @pl.when(jnp.logical_and(pp2 >= 0, pp2 < NP))
            def _():
                pltpu.make_async_copy(xb[h2], sh_sum.at[ib[h2]], scs[h2]).wait()
                pltpu.make_async_copy(ones, sh_cnt.at[ib[h2]], scs[h2]).wait()

            pn = p + NW

            @pl.when(pn < NP)
            def _():
                nbase = pn * P
                pltpu.async_copy(x_hbm.at[pl.ds(nbase, P)], xb[h2], ins[h2])
                pltpu.async_copy(b_hbm.at[pl.ds(nbase, P)], ib[h2], ins[h2])

        return 0

    lax.fori_loop(0, NITER // 3, _trip, 0)

    @pl.when(wid == NW - 1)
    def _():
        pltpu.sync_copy(x_hbm.at[pl.ds(TBASE, TAIL)], xbt)
        pltpu.sync_copy(b_hbm.at[pl.ds(TBASE, TAIL)], ibt)
        pltpu.sync_copy(xbt, sh_sum.at[ibt], add=True)
        pltpu.sync_copy(ones.at[pl.ds(0, TAIL)], sh_cnt.at[ibt], add=True)

    plsc.subcore_barrier()

    base = sid * SEG_W
    pltpu.sync_copy(sh_sum.at[pl.ds(base, SEG_W)], zbuf)
    pltpu.sync_copy(zbuf, sums_hbm.at[pl.ds(cid * S + base, SEG_W)])
    pltpu.sync_copy(sh_cnt.at[pl.ds(base, SEG_W)], zcbuf)
    pltpu.sync_copy(zcbuf, cnts_hbm.at[pl.ds(cid * S + base, SEG_W)])


@functools.partial(
    pl.kernel,
    out_type=jax.ShapeDtypeStruct((N, D), jnp.float32),
    mesh=_mesh,
    scratch_types=[
        pltpu.VMEM_SHARED((S, D), jnp.float32),
        pltpu.VMEM((SEG_W, D), jnp.float32),
        pltpu.VMEM((SEG_W, D), jnp.float32),
        pltpu.VMEM((SEG_W, D), jnp.float32),
        pltpu.VMEM((SEG_W, D), jnp.float32),
        [pltpu.VMEM((P, D), jnp.float32)] * 3,
        [pltpu.VMEM((P,), jnp.int32)] * 3,
        pltpu.VMEM((TAIL, D), jnp.float32),
        pltpu.VMEM((TAIL,), jnp.int32),
        [pltpu.SemaphoreType.DMA] * 3,
        [pltpu.SemaphoreType.DMA] * 3,
        [pltpu.SemaphoreType.DMA] * 3,
    ],
)
def _subtract(x_hbm, b_hbm, sums_hbm, cnts_hbm, y_hbm,
              sh_nmean, s0, s1, c0, c1, xb, ib, xbt, ibt, ins, gs, os):
    cid, sid, wid = _worker_ids()

    base = sid * SEG_W
    pltpu.sync_copy(sums_hbm.at[pl.ds(base, SEG_W)], s0)
    pltpu.sync_copy(sums_hbm.at[pl.ds(S + base, SEG_W)], s1)
    pltpu.sync_copy(cnts_hbm.at[pl.ds(base, SEG_W)], c0)
    pltpu.sync_copy(cnts_hbm.at[pl.ds(S + base, SEG_W)], c1)

    def _mrow(r, _):
        ds0 = pl.ds(0, 16)
        cnt = c0[r, ds0] + c1[r, ds0]
        ninv = jnp.float32(-1.0) / jnp.maximum(cnt, jnp.float32(1.0))
        for j in range(D // 16):
            ds = pl.ds(j * 16, 16)
            s0[r, ds] = (s0[r, ds] + s1[r, ds]) * ninv
        return 0

    lax.fori_loop(0, SEG_W, _mrow, 0)
    pltpu.sync_copy(s0, sh_nmean.at[pl.ds(base, SEG_W)])
    plsc.subcore_barrier()

    pltpu.async_copy(x_hbm.at[pl.ds(wid * P, P)], xb[0], ins[0])
    pltpu.async_copy(b_hbm.at[pl.ds(wid * P, P)], ib[0], ins[0])

    def _trip(kk, _):
        for h in range(3):
            k = kk * 3 + h
            p = k * NW + wid

            @pl.when(p < NP)
            def _():
                base = p * P
                pltpu.make_async_copy(x_hbm.at[pl.ds(base, P)], xb[h], ins[h]).wait()
                pltpu.make_async_copy(b_hbm.at[pl.ds(base, P)], ib[h], ins[h]).wait()
                pltpu.async_copy(sh_nmean.at[ib[h]], xb[h], gs[h], add=True)

            h1 = (h + 2) % 3
            pp1 = p - NW

            @pl.when(jnp.logical_and(pp1 >= 0, pp1 < NP))
            def _():
                pltpu.make_async_copy(sh_nmean.at[ib[h1]], xb[h1], gs[h1]).wait()
                pltpu.async_copy(xb[h1], y_hbm.at[pl.ds(pp1 * P, P)], os[h1])

            h2 = (h + 1) % 3
            pp2 = p - 2 * NW

            @pl.when(jnp.logical_and(pp2 >= 0, pp2 < NP))
            def _():
                pltpu.make_async_copy(xb[h2], y_hbm.at[pl.ds(pp2 * P, P)], os[h2]).wait()

            pn = p + NW

            @pl.when(pn < NP)
            def _():
                nbase = pn * P
                pltpu.async_copy(x_hbm.at[pl.ds(nbase, P)], xb[h2], ins[h2])
                pltpu.async_copy(b_hbm.at[pl.ds(nbase, P)], ib[h2], ins[h2])

        return 0

    lax.fori_loop(0, NITER // 3, _trip, 0)

    @pl.when(wid == NW - 1)
    def _():
        pltpu.sync_copy(x_hbm.at[pl.ds(TBASE, TAIL)], xbt)
        pltpu.sync_copy(b_hbm.at[pl.ds(TBASE, TAIL)], ibt)
        pltpu.sync_copy(sh_nmean.at[ibt], xbt, add=True)
        pltpu.sync_copy(xbt, y_hbm.at[pl.ds(TBASE, TAIL)])


def kernel(x, batch, dim_size):
    del dim_size
    b32 = batch.astype(jnp.int32)
    sums, cnts = _seg_sums(x, b32)
    return _subtract(x, b32, sums, cnts)
